# baseline (speedup 1.0000x reference)
import functools

import jax
import jax.numpy as jnp
from jax import lax
from jax.experimental import pallas as pl
from jax.experimental.pallas import tpu as pltpu

ROWS = 16384
COLS = 1024
BLK = 256
NBLK = ROWS // BLK


def _tc_body(x_ref, o_ref):
    nsl = COLS // 128
    xs = [x_ref[:, pl.ds(j * 128, 128)] for j in range(nsl)]
    m = xs[0]
    for j in range(1, nsl):
        m = jnp.maximum(m, xs[j])
    m = jnp.max(m, axis=-1, keepdims=True)
    li = lax.broadcasted_iota(jnp.int32, (BLK, 128), 1)
    cand = jnp.where(xs[0] == m, li, COLS)
    for j in range(1, nsl):
        cand = jnp.minimum(
            cand, jnp.where(xs[j] == m, li + j * 128, COLS)
        )
    o_ref[0, 0, :] = jnp.min(cand, axis=-1)


_argmax_tc = pl.pallas_call(
    _tc_body,
    grid=(NBLK,),
    in_specs=[pl.BlockSpec((BLK, COLS), lambda i: (i, 0))],
    out_specs=pl.BlockSpec((1, 1, BLK), lambda i: (i, 0, 0)),
    out_shape=jax.ShapeDtypeStruct((NBLK, 1, BLK), jnp.int32),
    compiler_params=pltpu.CompilerParams(
        dimension_semantics=("parallel",),
    ),
)


def kernel(emission, to_string):
    del to_string
    return _argmax_tc(emission).reshape(ROWS)


# speedup vs baseline: 1.6708x; 1.6708x over previous
import functools

import jax
import jax.numpy as jnp
from jax import lax
from jax.experimental import pallas as pl
from jax.experimental.pallas import tpu as pltpu

ROWS = 16384
COLS = 1024
BLK = 256
NBLK = ROWS // BLK
NBUF = 4


def _tc_body(em_hbm, o_hbm, buf, ov, osem, *sems):
    def blk_dma(g, b):
        return pltpu.make_async_copy(
            em_hbm.at[pl.ds(g * BLK, BLK), :], buf.at[b], sems[b]
        )

    for b in range(NBUF - 1):
        blk_dma(b, b).start()

    def compute(g, b):
        nsl = COLS // 128
        xs = [buf[b, :, pl.ds(j * 128, 128)] for j in range(nsl)]
        m = xs[0]
        for j in range(1, nsl):
            m = jnp.maximum(m, xs[j])
        m = jnp.max(m, axis=-1, keepdims=True)
        li = lax.broadcasted_iota(jnp.int32, (BLK, 128), 1)
        cand = jnp.where(xs[0] == m, li, COLS)
        for j in range(1, nsl):
            cand = jnp.minimum(cand, jnp.where(xs[j] == m, li + j * 128, COLS))
        ov[pl.ds(g * BLK, BLK)] = jnp.min(cand, axis=-1)

    def outer(i, _):
        g0 = NBUF * i
        for b in range(NBUF):
            g = g0 + b

            @pl.when(g + NBUF - 1 < NBLK)
            def _():
                blk_dma(g + NBUF - 1, (b + NBUF - 1) % NBUF).start()

            blk_dma(g, b).wait()
            compute(g, b)
        return 0

    lax.fori_loop(0, NBLK // NBUF, outer, 0)
    pltpu.make_async_copy(ov, o_hbm, osem).start()
    pltpu.make_async_copy(ov, o_hbm, osem).wait()


_argmax_tc = pl.pallas_call(
    _tc_body,
    in_specs=[pl.BlockSpec(memory_space=pl.ANY)],
    out_specs=pl.BlockSpec(memory_space=pl.ANY),
    out_shape=jax.ShapeDtypeStruct((ROWS,), jnp.int32),
    scratch_shapes=[
        pltpu.VMEM((NBUF, BLK, COLS), jnp.float32),
        pltpu.VMEM((ROWS,), jnp.int32),
        pltpu.SemaphoreType.DMA,
    ]
    + [pltpu.SemaphoreType.DMA] * NBUF,
)


def kernel(emission, to_string):
    del to_string
    return _argmax_tc(emission)
